# Initial kernel scaffold; baseline (speedup 1.0000x reference)
import jax
import jax.numpy as jnp
from jax.experimental import pallas as pl


def kernel(x, edge_index, W1, b1, W2, b2):
    raise NotImplementedError("write your pallas kernel here")


# speedup vs baseline: 12.8536x; 12.8536x over previous
import functools

import jax
import jax.numpy as jnp
from jax import lax
from jax.experimental import pallas as pl
from jax.experimental.pallas import tpu as pltpu
from jax.experimental.pallas import tpu_sc as plsc

N = 10000
NP = 10240
E = 320000
D = 128
NC = 2
NS = 16
NW = NC * NS
EPW = E // NW
K = 80
NCHUNK = EPW // K
RPT = NP // NS

_mesh = plsc.VectorSubcoreMesh(core_axis_name="c", subcore_axis_name="s")


@functools.partial(
    pl.kernel,
    out_type=jax.ShapeDtypeStruct((NC, NP), jnp.float32),
    mesh=_mesh,
    scratch_types=[
        pltpu.VMEM((K,), jnp.int32),
        pltpu.VMEM((K,), jnp.float32),
        pltpu.VMEM_SHARED((NP,), jnp.float32),
    ],
)
def _deg_partials(col_hbm, ones_hbm, zeros_hbm, out_hbm, colb, onesb, acc):
    c = lax.axis_index("c")
    s = lax.axis_index("s")
    wid = s * NC + c
    pltpu.sync_copy(ones_hbm, onesb)

    @pl.when(s == 0)
    def _():
        pltpu.sync_copy(zeros_hbm, acc)

    plsc.subcore_barrier()

    def body(j, carry):
        base = wid * EPW + j * K
        pltpu.sync_copy(col_hbm.at[pl.ds(base, K)], colb)
        pltpu.sync_copy(onesb, acc.at[colb], add=True)
        return carry

    lax.fori_loop(0, NCHUNK, body, 0)
    plsc.subcore_barrier()

    @pl.when(s == 0)
    def _():
        pltpu.sync_copy(acc, out_hbm.at[c])


@functools.partial(
    pl.kernel,
    out_type=jax.ShapeDtypeStruct((NC, NP, D), jnp.float32),
    mesh=_mesh,
    scratch_types=[
        pltpu.VMEM((K,), jnp.int32),
        pltpu.VMEM((K,), jnp.int32),
        pltpu.VMEM((K, D), jnp.float32),
        pltpu.VMEM_SHARED((NP, D), jnp.float32),
        pltpu.SemaphoreType.DMA,
    ],
)
def _scatter_partials(y_hbm, row_hbm, col_hbm, zeros_hbm, out_hbm,
                      rowb, colb, gbuf, acc, sem):
    c = lax.axis_index("c")
    s = lax.axis_index("s")
    wid = s * NC + c
    pltpu.sync_copy(zeros_hbm.at[pl.ds(s * RPT, RPT)], acc.at[pl.ds(s * RPT, RPT)])
    plsc.subcore_barrier()

    def body(j, carry):
        base = wid * EPW + j * K
        pltpu.sync_copy(row_hbm.at[pl.ds(base, K)], rowb)
        pltpu.sync_copy(col_hbm.at[pl.ds(base, K)], colb)
        pltpu.async_copy(y_hbm.at[rowb], gbuf, sem).wait()
        pltpu.sync_copy(gbuf, acc.at[colb], add=True)
        return carry

    lax.fori_loop(0, NCHUNK, body, 0)
    plsc.subcore_barrier()
    pltpu.sync_copy(acc.at[pl.ds(s * RPT, RPT)],
                    out_hbm.at[c, pl.ds(s * RPT, RPT)])


R = 1000


def _tcb_body(degt_ref, x_ref, w_ref, y_ref, disb_ref):
    d = degt_ref[...]
    dis = lax.rsqrt(d[:, 0:1] + d[:, 1:2] + 1.0)
    xw = jnp.dot(x_ref[...], w_ref[...], preferred_element_type=jnp.float32)
    y_ref[...] = dis * xw
    disb_ref[...] = jnp.broadcast_to(dis, xw.shape)


_tcb = pl.pallas_call(
    _tcb_body,
    grid=(N // R,),
    in_specs=[
        pl.BlockSpec((R, 2), lambda i: (i, 0)),
        pl.BlockSpec((R, D), lambda i: (i, 0)),
        pl.BlockSpec((D, D), lambda i: (0, 0)),
    ],
    out_specs=[
        pl.BlockSpec((R, D), lambda i: (i, 0)),
        pl.BlockSpec((R, D), lambda i: (i, 0)),
    ],
    out_shape=[
        jax.ShapeDtypeStruct((N, D), jnp.float32),
        jax.ShapeDtypeStruct((N, D), jnp.float32),
    ],
)


def _tcd_body(a0_ref, a1_ref, y1_ref, disb_ref, b1_ref, w2_ref, y2_ref):
    dis = disb_ref[...]
    h = jnp.tanh(dis * (a0_ref[...] + a1_ref[...] + y1_ref[...]) + b1_ref[...])
    y2_ref[...] = dis * jnp.dot(h, w2_ref[...], preferred_element_type=jnp.float32)


_tcd = pl.pallas_call(
    _tcd_body,
    grid=(N // R,),
    in_specs=[
        pl.BlockSpec((R, D), lambda i: (i, 0)),
        pl.BlockSpec((R, D), lambda i: (i, 0)),
        pl.BlockSpec((R, D), lambda i: (i, 0)),
        pl.BlockSpec((R, D), lambda i: (i, 0)),
        pl.BlockSpec((1, D), lambda i: (0, 0)),
        pl.BlockSpec((D, D), lambda i: (0, 0)),
    ],
    out_specs=pl.BlockSpec((R, D), lambda i: (i, 0)),
    out_shape=jax.ShapeDtypeStruct((N, D), jnp.float32),
)


def _tcf_body(a0_ref, a1_ref, y2_ref, disb_ref, b2_ref, o_ref):
    o_ref[...] = (disb_ref[...] * (a0_ref[...] + a1_ref[...] + y2_ref[...])
                  + b2_ref[...])


_tcf = pl.pallas_call(
    _tcf_body,
    grid=(N // R,),
    in_specs=[
        pl.BlockSpec((R, D), lambda i: (i, 0)),
        pl.BlockSpec((R, D), lambda i: (i, 0)),
        pl.BlockSpec((R, D), lambda i: (i, 0)),
        pl.BlockSpec((R, D), lambda i: (i, 0)),
        pl.BlockSpec((1, D), lambda i: (0, 0)),
    ],
    out_specs=pl.BlockSpec((R, D), lambda i: (i, 0)),
    out_shape=jax.ShapeDtypeStruct((N, D), jnp.float32),
)


@jax.jit
def kernel(x, edge_index, W1, b1, W2, b2):
    row = edge_index[0].astype(jnp.int32)
    col = edge_index[1].astype(jnp.int32)
    x = x.astype(jnp.float32)

    ones_k = jnp.ones((K,), jnp.float32)
    zeros1 = jnp.zeros((NP,), jnp.float32)
    zeros2 = jnp.zeros((NP, D), jnp.float32)

    degp = _deg_partials(col, ones_k, zeros1)
    degt = degp.T

    y1, disb = _tcb(degt, x, W1)
    a1 = _scatter_partials(y1, row, col, zeros2)
    y2 = _tcd(a1[0], a1[1], y1, disb, b1.reshape(1, D), W2)
    a2 = _scatter_partials(y2, row, col, zeros2)
    out = _tcf(a2[0], a2[1], y2, disb, b2.reshape(1, D))
    return out
